# baseline (speedup 1.0000x reference)
import functools
import math

import jax
import jax.numpy as jnp
from jax.experimental import pallas as pl
from jax.experimental.pallas import tpu as pltpu

N_CONV = 3
FEAT = 128
N_RBF = 20
CUTOFF = 5.0


def _phi_body(h_ref, w1_ref, b1_ref, w2_ref, b2_ref, out_ref):
    h = h_ref[...]
    z = jnp.dot(h, w1_ref[...], preferred_element_type=jnp.float32) + b1_ref[...]
    z = z * jax.nn.sigmoid(z)
    out_ref[...] = (
        jnp.dot(z, w2_ref[...], preferred_element_type=jnp.float32) + b2_ref[...]
    )


def _phi(h, W1, b1, W2, b2, block=1000):
    n = h.shape[0]
    grid = (n // block,)
    return pl.pallas_call(
        _phi_body,
        grid=grid,
        in_specs=[
            pl.BlockSpec((block, FEAT), lambda i: (i, 0)),
            pl.BlockSpec((FEAT, FEAT), lambda i: (0, 0)),
            pl.BlockSpec((1, FEAT), lambda i: (0, 0)),
            pl.BlockSpec((FEAT, 3 * FEAT), lambda i: (0, 0)),
            pl.BlockSpec((1, 3 * FEAT), lambda i: (0, 0)),
        ],
        out_specs=pl.BlockSpec((block, 3 * FEAT), lambda i: (i, 0)),
        out_shape=jax.ShapeDtypeStruct((n, 3 * FEAT), jnp.float32),
    )(h, W1, b1.reshape(1, -1), W2, b2.reshape(1, -1))


def _ws_body(rbf_ref, env_ref, wd_ref, bd_ref, out_ref):
    ws = jnp.dot(rbf_ref[...], wd_ref[...], preferred_element_type=jnp.float32)
    out_ref[...] = (ws + bd_ref[...]) * env_ref[...]


def _ws(rbf, env, Wd, bd, block=4000):
    e = rbf.shape[0]
    grid = (e // block,)
    return pl.pallas_call(
        _ws_body,
        grid=grid,
        in_specs=[
            pl.BlockSpec((block, N_RBF), lambda i: (i, 0)),
            pl.BlockSpec((block, 1), lambda i: (i, 0)),
            pl.BlockSpec((N_RBF, 3 * FEAT), lambda i: (0, 0)),
            pl.BlockSpec((1, 3 * FEAT), lambda i: (0, 0)),
        ],
        out_specs=pl.BlockSpec((block, 3 * FEAT), lambda i: (i, 0)),
        out_shape=jax.ShapeDtypeStruct((e, 3 * FEAT), jnp.float32),
    )(rbf, env, Wd, bd.reshape(1, -1))


def _head_body(h_ref, w1_ref, b1_ref, w2_ref, b2_ref, out_ref):
    z = jnp.tanh(
        jnp.dot(h_ref[...], w1_ref[...], preferred_element_type=jnp.float32)
        + b1_ref[...]
    )
    out_ref[...] = (
        jnp.dot(z, w2_ref[...], preferred_element_type=jnp.float32) + b2_ref[...]
    )


def _head(h, W1, b1, W2, b2, block=1000):
    n = h.shape[0]
    return pl.pallas_call(
        _head_body,
        grid=(n // block,),
        in_specs=[
            pl.BlockSpec((block, FEAT), lambda i: (i, 0)),
            pl.BlockSpec((FEAT, FEAT), lambda i: (0, 0)),
            pl.BlockSpec((1, FEAT), lambda i: (0, 0)),
            pl.BlockSpec((FEAT, FEAT), lambda i: (0, 0)),
            pl.BlockSpec((1, FEAT), lambda i: (0, 0)),
        ],
        out_specs=pl.BlockSpec((block, FEAT), lambda i: (i, 0)),
        out_shape=jax.ShapeDtypeStruct((n, FEAT), jnp.float32),
    )(h, W1, b1.reshape(1, -1), W2, b2.reshape(1, -1))


def kernel(cg_z, cg_xyz, cg_nbr_list, emb, msg_W1, msg_b1, msg_W2, msg_b2,
           msg_Wd, msg_bd, mu_W1, mu_b1, mu_W2, mu_b2, sig_W1, sig_b1,
           sig_W2, sig_b2):
    E = cg_nbr_list.shape[0]
    N = cg_z.shape[0]

    gtr_ij = (cg_nbr_list[:, 0] > cg_nbr_list[:, 1]).any()
    gtr_ji = (cg_nbr_list[:, 1] > cg_nbr_list[:, 0]).any()
    directed = jnp.logical_and(gtr_ij, gtr_ji)
    nbrs = jnp.concatenate([cg_nbr_list, cg_nbr_list[:, ::-1]], axis=0)
    rev_w = jnp.where(directed, 0.0, 1.0).astype(jnp.float32)
    edge_w = jnp.concatenate(
        [jnp.ones((E,), jnp.float32), jnp.full((E,), 1.0) * rev_w], axis=0)

    h = emb[cg_z]
    v = jnp.zeros((N, FEAT, 3), dtype=jnp.float32)
    src = nbrs[:, 0]
    dst = nbrs[:, 1]
    r_ij = cg_xyz[dst] - cg_xyz[src]
    dist = jnp.sqrt((r_ij ** 2).sum(-1) + 1e-15)
    unit = r_ij / dist[:, None]
    nvec = jnp.arange(1, N_RBF + 1, dtype=jnp.float32)
    rbf = jnp.sin((nvec * math.pi / CUTOFF) * dist[:, None]) / dist[:, None]
    env = jnp.where(dist < CUTOFF, 0.5 * (jnp.cos(math.pi * dist / CUTOFF) + 1.0), 0.0)

    for i in range(N_CONV):
        phi = _phi(h, msg_W1[i], msg_b1[i], msg_W2[i], msg_b2[i])
        w_s = _ws(rbf, env[:, None], msg_Wd[i], msg_bd[i])
        inv_out = (phi[dst] * w_s).reshape(-1, 3, FEAT)
        split_0 = inv_out[:, 0, :][..., None]
        split_1 = inv_out[:, 1, :]
        split_2 = inv_out[:, 2, :][..., None]
        dv_ij = split_2 * unit[:, None, :] + split_0 * v[dst]
        h = h + jax.ops.segment_sum(split_1 * edge_w[:, None], src, num_segments=N)
        v = v + jax.ops.segment_sum(dv_ij * edge_w[:, None, None], src,
                                    num_segments=N)

    H_mu = _head(h, mu_W1, mu_b1, mu_W2, mu_b2)
    H_sigma = _head(h, sig_W1, sig_b1, sig_W2, sig_b2)
    H_std = 1e-09 + jnp.exp(H_sigma / 2)
    return (H_mu, H_std)
